# Initial kernel scaffold; baseline (speedup 1.0000x reference)
import jax
import jax.numpy as jnp
from jax.experimental import pallas as pl


def kernel(boxes, scores):
    raise NotImplementedError("write your pallas kernel here")


# speedup vs baseline: 25.7472x; 25.7472x over previous
import jax
import jax.numpy as jnp
from jax.experimental import pallas as pl
from jax.experimental.pallas import tpu as pltpu

_PRE = 2000
_POST = 1000
_PAD = 2048
_BLK = 256
_NBLK = _PAD // _BLK
_THR = 0.7


def _nms_body(xc_ref, yc_ref, xxc_ref, yyc_ref,
              xr_ref, yr_ref, xxr_ref, yyr_ref,
              keep_ref, over_ref):
    col = jax.lax.broadcasted_iota(jnp.int32, (1, _PAD), 1)
    keep_ref[...] = (col < _PRE).astype(jnp.float32)

    xc = xc_ref[...]
    yc = yc_ref[...]
    xxc = xxc_ref[...]
    yyc = yyc_ref[...]
    area_c = (xxc - xc) * (yyc - yc)
    lane = jax.lax.broadcasted_iota(jnp.int32, (1, _BLK), 1)

    def blk_body(b, carry):
        base = b * _BLK
        x = xr_ref[pl.ds(base, _BLK), :]
        y = yr_ref[pl.ds(base, _BLK), :]
        xx = xxr_ref[pl.ds(base, _BLK), :]
        yy = yyr_ref[pl.ds(base, _BLK), :]
        area_r = (xx - x) * (yy - y)
        iw = jnp.maximum(jnp.minimum(xx, xxc) - jnp.maximum(x, xc), 0.0)
        ih = jnp.maximum(jnp.minimum(yy, yyc) - jnp.maximum(y, yc), 0.0)
        inter = iw * ih
        union = jnp.maximum(area_r + area_c - inter, 1e-9)
        over = (inter / union > _THR).astype(jnp.float32)
        over_ref[...] = over

        def inner(i, kblk):
            row = over_ref[pl.ds(i, 1), pl.ds(base, _BLK)]
            ki = jnp.sum(kblk * (lane == i).astype(jnp.float32))
            sup = row * ki * (lane > i).astype(jnp.float32)
            return kblk * (1.0 - sup)

        kblk = jax.lax.fori_loop(0, _BLK, inner,
                                 keep_ref[:, pl.ds(base, _BLK)])
        keep_ref[:, pl.ds(base, _BLK)] = kblk

        later = jnp.dot(kblk, over, preferred_element_type=jnp.float32)
        hit = (later > 0.5).astype(jnp.float32)
        mask = (col >= base + _BLK).astype(jnp.float32)
        keep_ref[...] = keep_ref[...] * (1.0 - hit * mask)
        return carry

    jax.lax.fori_loop(0, _NBLK, blk_body, 0)


@jax.jit
def _run_nms(bx):
    xc = bx[:, 0].reshape(1, _PAD)
    yc = bx[:, 1].reshape(1, _PAD)
    xxc = bx[:, 2].reshape(1, _PAD)
    yyc = bx[:, 3].reshape(1, _PAD)
    xr = bx[:, 0].reshape(_PAD, 1)
    yr = bx[:, 1].reshape(_PAD, 1)
    xxr = bx[:, 2].reshape(_PAD, 1)
    yyr = bx[:, 3].reshape(_PAD, 1)
    return pl.pallas_call(
        _nms_body,
        out_shape=jax.ShapeDtypeStruct((1, _PAD), jnp.float32),
        scratch_shapes=[pltpu.VMEM((_BLK, _PAD), jnp.float32)],
    )(xc, yc, xxc, yyc, xr, yr, xxr, yyr)


def kernel(boxes, scores):
    w = boxes[:, 2] - boxes[:, 0]
    h = boxes[:, 3] - boxes[:, 1]
    valid = (w >= 0.0) & (h >= 0.0)
    scores_f = jnp.where(valid, scores, -jnp.inf)

    top_scores, top_idx = jax.lax.top_k(scores_f, _PRE)
    b = boxes[top_idx]

    bx = jnp.zeros((_PAD, 4), jnp.float32).at[:_PRE].set(b)
    keep = _run_nms(bx)[0, :_PRE] > 0.5

    idx = jnp.arange(_PRE, dtype=jnp.int32)
    order_key = jnp.where(keep, idx, idx + _PRE)
    sel = jnp.argsort(order_key)[:_POST]

    out_boxes = b[sel]
    out_scores = top_scores[sel]
    return jnp.concatenate([out_boxes, out_scores[:, None]], axis=-1)
